# baseline (speedup 1.0000x reference)
import functools

import jax
import jax.numpy as jnp
from jax import lax
from jax.experimental import pallas as pl
from jax.experimental.pallas import tpu as pltpu
from jax.experimental.pallas import tpu_sc as plsc

EPS = 1e-12
L = 16

_GATHER_DNUMS = lax.GatherDimensionNumbers(
    offset_dims=(), collapsed_slice_dims=(0,), start_index_map=(0,))


def _shuffle(v, p):
    return lax.gather(v, p[:, None], dimension_numbers=_GATHER_DNUMS,
                      slice_sizes=(1,),
                      mode=lax.GatherScatterMode.PROMISE_IN_BOUNDS)


def _allsum(v, perms):
    for p in perms:
        v = v + _shuffle(v, p)
    return v


def _make_sc_kernel(N, E, S, NC, NS):
    NW = NC * NS
    TPW = N // NW
    NCH = TPW // 128
    KV = E // L

    mesh = plsc.VectorSubcoreMesh(core_axis_name="c", subcore_axis_name="s")

    @functools.partial(
        pl.kernel,
        mesh=mesh,
        out_type=jax.ShapeDtypeStruct((N, E), jnp.float32),
        scratch_types=[
            pltpu.VMEM((NCH, 128), jnp.int32),
            pltpu.VMEM((TPW,), jnp.int32),
            pltpu.VMEM((TPW, E), jnp.float32),
            pltpu.VMEM((TPW, E), jnp.float32),
            pltpu.VMEM((2, E), jnp.float32),
            pltpu.VMEM((2, E), jnp.float32),
            pltpu.SemaphoreType.DMA,
            pltpu.SemaphoreType.DMA,
        ],
    )
    def body(ids_h, tt_h, word_h, type_h, pos_h, gb_h, out_h,
             idx_v, tt_v, rows_v, pos_v, ty_v, gb_v, gsem0, gsem1):
        wid = lax.axis_index("s") * NC + lax.axis_index("c")
        base = wid * TPW
        pbase = lax.rem(base, S)

        pltpu.sync_copy(ids_h.at[wid], idx_v)
        gsems = [gsem0, gsem1]
        gathers = [pltpu.async_copy(
            word_h.at[idx_v.at[c]], rows_v.at[pl.ds(c * 128, 128)], gsems[c])
            for c in range(NCH)]
        pltpu.sync_copy(tt_h.at[wid], tt_v)
        pltpu.sync_copy(pos_h.at[pl.ds(pbase, TPW)], pos_v)
        pltpu.sync_copy(type_h, ty_v)
        pltpu.sync_copy(gb_h, gb_v)

        t0 = [ty_v[0, pl.ds(L * k, L)] for k in range(KV)]
        td = [ty_v[1, pl.ds(L * k, L)] - t0[k] for k in range(KV)]
        gvec = [gb_v[0, pl.ds(L * k, L)] for k in range(KV)]
        bvec = [gb_v[1, pl.ds(L * k, L)] for k in range(KV)]
        iota = lax.iota(jnp.int32, L)
        perms = [iota ^ 1, iota ^ 2, iota ^ 4, iota ^ 8]
        inv_e = jnp.float32(1.0 / E)

        def group_fn(g, carry):
            rbase = g * L
            tslice = tt_v[pl.ds(rbase, L)].astype(jnp.float32)
            for j in range(L):
                r = rbase + j
                ttf = _shuffle(tslice, lax.broadcast(jnp.int32(j), (L,)))
                cv = [rows_v[r, pl.ds(L * k, L)]
                      + pos_v[r, pl.ds(L * k, L)]
                      + (t0[k] + ttf * td[k]) for k in range(KV)]
                s = (cv[0] + cv[1]) + (cv[2] + cv[3])
                s = s + ((cv[4] + cv[5]) + (cv[6] + cv[7]))
                q = cv[0] * cv[0]
                for k in range(1, KV):
                    q = q + cv[k] * cv[k]
                mean = _allsum(s, perms) * inv_e
                var = _allsum(q, perms) * inv_e - mean * mean
                x = var + jnp.float32(EPS)
                i = lax.bitcast_convert_type(x, jnp.int32)
                i = jnp.int32(0x5F3759DF) - lax.shift_right_arithmetic(i, 1)
                y = lax.bitcast_convert_type(i, jnp.float32)
                for _ in range(2):
                    y = y * (jnp.float32(1.5) - jnp.float32(0.5) * x * y * y)
                for k in range(KV):
                    rows_v[r, pl.ds(L * k, L)] = (cv[k] - mean) * y * gvec[k] + bvec[k]
            return carry

        gpc = 128 // L
        for c in range(NCH):
            gathers[c].wait()
            lax.fori_loop(c * gpc, (c + 1) * gpc, group_fn, 0)
        pltpu.sync_copy(rows_v, out_h.at[pl.ds(base, TPW)])

    return body


def kernel(input_ids, token_type_ids, word_emb, type_emb, pos_emb,
           ln_gamma, ln_beta):
    B, S = input_ids.shape
    V, E = word_emb.shape
    N = B * S

    info = plsc.get_sparse_core_info()
    NC, NS = info.num_cores, info.num_subcores
    NW = NC * NS

    ids2 = input_ids.reshape(NW, N // NW // 128, 128).astype(jnp.int32)
    tt2 = token_type_ids.reshape(NW, N // NW).astype(jnp.int32)
    gb = jnp.stack([ln_gamma, ln_beta]).astype(jnp.float32)

    sc = _make_sc_kernel(N, E, S, NC, NS)
    out = sc(ids2, tt2, word_emb, type_emb, pos_emb, gb)
    return out.reshape(B, S, E)


# speedup vs baseline: 1.2133x; 1.2133x over previous
import functools

import jax
import jax.numpy as jnp
from jax import lax
from jax.experimental import pallas as pl
from jax.experimental.pallas import tpu as pltpu
from jax.experimental.pallas import tpu_sc as plsc

EPS = 1e-12
L = 16

_GATHER_DNUMS = lax.GatherDimensionNumbers(
    offset_dims=(), collapsed_slice_dims=(0,), start_index_map=(0,))


def _shuffle(v, p):
    return lax.gather(v, p[:, None], dimension_numbers=_GATHER_DNUMS,
                      slice_sizes=(1,),
                      mode=lax.GatherScatterMode.PROMISE_IN_BOUNDS)


def _allsum(v, perms):
    for p in perms:
        v = v + _shuffle(v, p)
    return v


def _make_sc_kernel(N, E, S, NC, NS):
    NW = NC * NS
    TPW = N // NW
    NCH = TPW // 128
    KV = E // L

    mesh = plsc.VectorSubcoreMesh(core_axis_name="c", subcore_axis_name="s")

    @functools.partial(
        pl.kernel,
        mesh=mesh,
        out_type=jax.ShapeDtypeStruct((N, E), jnp.float32),
        scratch_types=[
            pltpu.VMEM((2, NCH, 128), jnp.int32),
            pltpu.VMEM((TPW, E), jnp.float32),
            pltpu.VMEM((TPW, E), jnp.float32),
            pltpu.VMEM((4, E), jnp.float32),
            pltpu.SemaphoreType.DMA,
        ],
    )
    def body(it_h, word_h, tygb_h, pos_h, out_h,
             it_v, rows_v, pos_v, tygb_v, gsem0):
        wid = lax.axis_index("s") * NC + lax.axis_index("c")
        base = wid * TPW
        pbase = lax.rem(base, S)

        pltpu.sync_copy(it_h.at[wid], it_v)
        gathers = [pltpu.async_copy(
            word_h.at[it_v.at[0, c]], rows_v.at[pl.ds(c * 128, 128)], gsem0)
            for c in range(NCH)]
        pltpu.sync_copy(pos_h.at[pl.ds(pbase, TPW)], pos_v)
        pltpu.sync_copy(tygb_h, tygb_v)

        t0 = [tygb_v[0, pl.ds(L * k, L)] for k in range(KV)]
        td = [tygb_v[1, pl.ds(L * k, L)] - t0[k] for k in range(KV)]
        gvec = [tygb_v[2, pl.ds(L * k, L)] for k in range(KV)]
        bvec = [tygb_v[3, pl.ds(L * k, L)] for k in range(KV)]
        iota = lax.iota(jnp.int32, L)
        perms = [iota ^ 1, iota ^ 2, iota ^ 4, iota ^ 8]
        inv_e = jnp.float32(1.0 / E)

        def group_fn(g, carry):
            rbase = g * L
            tslice = it_v[1, g // (128 // L),
                          pl.ds(lax.rem(g, 128 // L) * L, L)].astype(jnp.float32)
            for j in range(L):
                r = rbase + j
                ttf = _shuffle(tslice, lax.broadcast(jnp.int32(j), (L,)))
                cv = [rows_v[r, pl.ds(L * k, L)]
                      + pos_v[r, pl.ds(L * k, L)]
                      + (t0[k] + ttf * td[k]) for k in range(KV)]
                s = (cv[0] + cv[1]) + (cv[2] + cv[3])
                s = s + ((cv[4] + cv[5]) + (cv[6] + cv[7]))
                q = cv[0] * cv[0]
                for k in range(1, KV):
                    q = q + cv[k] * cv[k]
                mean = _allsum(s, perms) * inv_e
                var = _allsum(q, perms) * inv_e - mean * mean
                x = var + jnp.float32(EPS)
                i = lax.bitcast_convert_type(x, jnp.int32)
                i = jnp.int32(0x5F3759DF) - lax.shift_right_arithmetic(i, 1)
                y = lax.bitcast_convert_type(i, jnp.float32)
                for _ in range(2):
                    y = y * (jnp.float32(1.5) - jnp.float32(0.5) * x * y * y)
                for k in range(KV):
                    rows_v[r, pl.ds(L * k, L)] = (cv[k] - mean) * y * gvec[k] + bvec[k]
            return carry

        for cp in gathers:
            cp.wait()
        lax.fori_loop(0, TPW // L, group_fn, 0)
        pltpu.sync_copy(rows_v, out_h.at[pl.ds(base, TPW)])

    return body


def kernel(input_ids, token_type_ids, word_emb, type_emb, pos_emb,
           ln_gamma, ln_beta):
    B, S = input_ids.shape
    V, E = word_emb.shape
    N = B * S

    info = plsc.get_sparse_core_info()
    NC, NS = info.num_cores, info.num_subcores
    NW = NC * NS

    NCH = N // NW // 128
    it = jnp.stack([
        input_ids.reshape(NW, NCH, 128).astype(jnp.int32),
        token_type_ids.reshape(NW, NCH, 128).astype(jnp.int32)], axis=1)
    tygb = jnp.concatenate([
        type_emb.astype(jnp.float32),
        jnp.stack([ln_gamma, ln_beta]).astype(jnp.float32)])

    sc = _make_sc_kernel(N, E, S, NC, NS)
    out = sc(it, word_emb, tygb, pos_emb)
    return out.reshape(B, S, E)
